# baseline (speedup 1.0000x reference)
import functools

import jax
import jax.numpy as jnp
from jax import lax
from jax.experimental import pallas as pl
from jax.experimental.pallas import tpu as pltpu
from jax.experimental.pallas import tpu_sc as plsc

_N_E = 8192
_E_DIM = 256
_BETA = 0.25
_TB = 256
_NT = _N_E // _TB
_CBK = 2048
_NCK = _N_E // _CBK


def _dist_argmin_body(z_ref, wt_ref, sz_ref, sw_ref,
                      idx_ref, loss_ref, uniq_ref,
                      pres_acc, loss_acc):
    i = pl.program_id(0)

    @pl.when(i == 0)
    def _init():
        loss_acc[0, 0] = jnp.float32(0.0)
        pres_acc[...] = jnp.zeros((1, _N_E), jnp.int32)

    zb = z_ref[...]
    rmin = jnp.full((_TB, 1), jnp.inf, jnp.float32)
    ridx = jnp.zeros((_TB, 1), jnp.int32)
    for j in range(_NCK):
        wt_j = wt_ref[:, pl.ds(j * _CBK, _CBK)]
        dot = lax.dot_general(zb, wt_j, (((1,), (0,)), ((), ())),
                              preferred_element_type=jnp.float32)
        sw_j = sw_ref[:, pl.ds(j * _CBK, _CBK)]
        d = (sz_ref[...] + sw_j) - 2.0 * dot
        cmin = jnp.min(d, axis=1, keepdims=True)
        ci = lax.broadcasted_iota(jnp.int32, (_TB, _CBK), 1) + jnp.int32(j * _CBK)
        cidx = jnp.min(jnp.where(d == cmin, ci, jnp.int32(_N_E)),
                       axis=1, keepdims=True)
        better = cmin < rmin
        rmin = jnp.where(better, cmin, rmin)
        ridx = jnp.where(better, cidx, ridx)

    idx_ref[...] = ridx
    loss_acc[0, 0] += jnp.sum(rmin)
    for j in range(_NCK):
        ci = lax.broadcasted_iota(jnp.int32, (_TB, _CBK), 1) + jnp.int32(j * _CBK)
        pres = jnp.any(ridx == ci, axis=0, keepdims=True)
        pres_acc[:, pl.ds(j * _CBK, _CBK)] |= pres.astype(jnp.int32)

    @pl.when(i == _NT - 1)
    def _fini():
        loss_ref[...] = jnp.full((1, 1), loss_acc[0, 0], jnp.float32)
        uniq_ref[...] = jnp.sum(pres_acc[...], keepdims=True)


def _dist_argmin(z_flat, wt, sz, sw):
    return pl.pallas_call(
        _dist_argmin_body,
        grid=(_NT,),
        in_specs=[
            pl.BlockSpec((_TB, _E_DIM), lambda i: (i, 0)),
            pl.BlockSpec((_E_DIM, _N_E), lambda i: (0, 0)),
            pl.BlockSpec((_TB, 1), lambda i: (i, 0)),
            pl.BlockSpec((1, _N_E), lambda i: (0, 0)),
        ],
        out_specs=[
            pl.BlockSpec((_TB, 1), lambda i: (i, 0)),
            pl.BlockSpec((1, 1), lambda i: (0, 0)),
            pl.BlockSpec((1, 1), lambda i: (0, 0)),
        ],
        out_shape=[
            jax.ShapeDtypeStruct((_N_E, 1), jnp.int32),
            jax.ShapeDtypeStruct((1, 1), jnp.float32),
            jax.ShapeDtypeStruct((1, 1), jnp.int32),
        ],
        scratch_shapes=[
            pltpu.VMEM((1, _N_E), jnp.int32),
            pltpu.SMEM((1, 1), jnp.float32),
        ],
    )(z_flat, wt, sz, sw)


_SC_INFO = plsc.get_sparse_core_info()
_NW = _SC_INFO.num_cores * _SC_INFO.num_subcores
_BPW = _N_E // _NW
_CH = 128
_NCH = _BPW // _CH


def _sc_gather_body(table_hbm, idx_hbm, out_hbm, idx_a, idx_b, rows_v, sem):
    wid = lax.axis_index("s") * _SC_INFO.num_cores + lax.axis_index("c")
    base = wid * _BPW
    pltpu.sync_copy(idx_hbm.at[pl.ds(base, _CH)], idx_a)
    pltpu.sync_copy(idx_hbm.at[pl.ds(base + _CH, _CH)], idx_b)
    cp1 = pltpu.async_copy(table_hbm.at[idx_a], rows_v.at[pl.ds(0, _CH)], sem)
    cp2 = pltpu.async_copy(table_hbm.at[idx_b], rows_v.at[pl.ds(_CH, _CH)], sem)
    cp1.wait()
    cp2.wait()
    pltpu.sync_copy(rows_v, out_hbm.at[pl.ds(base, _BPW)])


_sc_gather = functools.partial(
    pl.kernel,
    mesh=plsc.VectorSubcoreMesh(core_axis_name="c", subcore_axis_name="s"),
    out_type=jax.ShapeDtypeStruct((_N_E, _E_DIM), jnp.float32),
    scratch_types=[
        pltpu.VMEM((_CH,), jnp.int32),
        pltpu.VMEM((_CH,), jnp.int32),
        pltpu.VMEM((_BPW, _E_DIM), jnp.float32),
        pltpu.SemaphoreType.DMA,
    ],
)(_sc_gather_body)


def kernel(z, W):
    zp = jnp.transpose(z, (0, 2, 3, 4, 1))
    z_flat = zp.reshape(-1, _E_DIM)
    sz = jnp.sum(z_flat ** 2, axis=1, keepdims=True)
    sw = jnp.sum(W ** 2, axis=1)
    idx2, loss_sum, uniq = _dist_argmin(z_flat, W.T, sz, sw.reshape(1, _N_E))
    idx = idx2.reshape(_N_E)

    zq_rows = _sc_gather(W, idx)
    zq = zq_rows.reshape(zp.shape)

    n_elems = z_flat.shape[0] * _E_DIM
    loss = (_BETA * (loss_sum[0, 0] / n_elems)) + (loss_sum[0, 0] / n_elems)
    z_q = zp + (zq - zp)
    z_q = jnp.transpose(z_q, (0, 4, 1, 2, 3))
    return (z_q, loss, uniq[0, 0], idx)


# speedup vs baseline: 1.2234x; 1.2234x over previous
import functools

import jax
import jax.numpy as jnp
from jax import lax
from jax.experimental import pallas as pl
from jax.experimental.pallas import tpu as pltpu
from jax.experimental.pallas import tpu_sc as plsc

_N_E = 8192
_E_DIM = 256
_BETA = 0.25
_TB = 256
_NT = _N_E // _TB


def _dist_argmin_body(z_ref, w_ref, sz_ref, sw_ref,
                      idx_ref, loss_ref, loss_acc):
    i = pl.program_id(0)

    @pl.when(i == 0)
    def _init():
        loss_acc[0, 0] = jnp.float32(0.0)

    zb = z_ref[...]
    zb2 = zb + zb
    dot2 = lax.dot_general(zb2, w_ref[...], (((1,), (0,)), ((), ())),
                           preferred_element_type=jnp.float32)
    d = (sz_ref[...] + sw_ref[...]) - dot2
    rmin = jnp.min(d, axis=1, keepdims=True)
    ci = lax.broadcasted_iota(jnp.int32, (_TB, _N_E), 1)
    ridx = jnp.min(jnp.where(d == rmin, ci, jnp.int32(_N_E)),
                   axis=1, keepdims=True)

    idx_ref[...] = ridx
    loss_acc[0, 0] += jnp.sum(rmin)

    @pl.when(i == _NT - 1)
    def _fini():
        loss_ref[...] = jnp.full((1, 1), loss_acc[0, 0], jnp.float32)


def _dist_argmin(z_flat, wt, sz, sw):
    return pl.pallas_call(
        _dist_argmin_body,
        grid=(_NT,),
        in_specs=[
            pl.BlockSpec((_TB, _E_DIM), lambda i: (i, 0)),
            pl.BlockSpec((_E_DIM, _N_E), lambda i: (0, 0)),
            pl.BlockSpec((_TB, 1), lambda i: (i, 0)),
            pl.BlockSpec((1, _N_E), lambda i: (0, 0)),
        ],
        out_specs=[
            pl.BlockSpec((_TB, 1), lambda i: (i, 0)),
            pl.BlockSpec((1, 1), lambda i: (0, 0)),
        ],
        out_shape=[
            jax.ShapeDtypeStruct((_N_E, 1), jnp.int32),
            jax.ShapeDtypeStruct((1, 1), jnp.float32),
        ],
        scratch_shapes=[
            pltpu.SMEM((1, 1), jnp.float32),
        ],
    )(z_flat, wt, sz, sw)


_SC_INFO = plsc.get_sparse_core_info()
_NC = _SC_INFO.num_cores
_NS = _SC_INFO.num_subcores
_NW = _NC * _NS
_BPW = _N_E // _NW
_CH = 128
_PC = _N_E // _NS
_L = 16


def _sc_gather_body(table_hbm, idx_hbm, out_hbm, pres_hbm,
                    idx_a, idx_b, rows_v, idx_w, ones_v, zeros_v,
                    shared_pres, sem, sem2):
    cid = lax.axis_index("c")
    sid = lax.axis_index("s")
    wid = sid * _NC + cid
    base = wid * _BPW
    pltpu.sync_copy(idx_hbm.at[pl.ds(base, _CH)], idx_a)
    pltpu.sync_copy(idx_hbm.at[pl.ds(base + _CH, _CH)], idx_b)
    cp1 = pltpu.async_copy(table_hbm.at[idx_a], rows_v.at[pl.ds(0, _CH)], sem)
    cp2 = pltpu.async_copy(table_hbm.at[idx_b], rows_v.at[pl.ds(_CH, _CH)], sem)

    @pl.when(cid == 0)
    def _prep():
        for k in range(_PC // _L):
            zeros_v[pl.ds(k * _L, _L)] = jnp.zeros((_L,), jnp.int32)
        pltpu.sync_copy(zeros_v, shared_pres.at[pl.ds(sid * _PC, _PC)])
        for r in range(4):
            pltpu.sync_copy(idx_hbm.at[pl.ds(sid * _PC + r * _CH, _CH)],
                            idx_w.at[r])
        for k in range(_CH // _L):
            ones_v[pl.ds(k * _L, _L)] = jnp.ones((_L,), jnp.int32)

    plsc.subcore_barrier()

    @pl.when(cid == 0)
    def _scatter():
        cps = [pltpu.async_copy(ones_v, shared_pres.at[idx_w.at[r]], sem2,
                                add=True) for r in range(4)]
        for cp in cps:
            cp.wait()

    plsc.subcore_barrier()

    @pl.when(cid == 0)
    def _dump():
        pltpu.sync_copy(shared_pres.at[pl.ds(sid * _PC, _PC)],
                        pres_hbm.at[pl.ds(sid * _PC, _PC)])

    cp1.wait()
    cp2.wait()
    pltpu.sync_copy(rows_v, out_hbm.at[pl.ds(base, _BPW)])


_sc_gather = functools.partial(
    pl.kernel,
    mesh=plsc.VectorSubcoreMesh(core_axis_name="c", subcore_axis_name="s"),
    out_type=[
        jax.ShapeDtypeStruct((_N_E, _E_DIM), jnp.float32),
        jax.ShapeDtypeStruct((_N_E,), jnp.int32),
    ],
    scratch_types=[
        pltpu.VMEM((_CH,), jnp.int32),
        pltpu.VMEM((_CH,), jnp.int32),
        pltpu.VMEM((_BPW, _E_DIM), jnp.float32),
        pltpu.VMEM((4, _CH), jnp.int32),
        pltpu.VMEM((_CH,), jnp.int32),
        pltpu.VMEM((_PC,), jnp.int32),
        pltpu.VMEM_SHARED((_N_E,), jnp.int32),
        pltpu.SemaphoreType.DMA,
        pltpu.SemaphoreType.DMA,
    ],
)(_sc_gather_body)


def _uniq_body(pres_ref, uniq_ref):
    uniq_ref[...] = jnp.sum(jnp.minimum(pres_ref[...], 1), keepdims=True)


def _uniq_count(pres2d):
    return pl.pallas_call(
        _uniq_body,
        out_shape=jax.ShapeDtypeStruct((1, 1), jnp.int32),
    )(pres2d)


def kernel(z, W):
    zp = jnp.transpose(z, (0, 2, 3, 4, 1))
    z_flat = zp.reshape(-1, _E_DIM)
    sz = jnp.sum(z_flat ** 2, axis=1, keepdims=True)
    sw = jnp.sum(W ** 2, axis=1)
    idx2, loss_sum = _dist_argmin(z_flat, W.T, sz, sw.reshape(1, _N_E))
    idx = idx2.reshape(_N_E)

    zq_rows, pres = _sc_gather(W, idx)
    zq = zq_rows.reshape(zp.shape)
    uniq = _uniq_count(pres.reshape(1, _N_E))

    n_elems = z_flat.shape[0] * _E_DIM
    loss = (_BETA * (loss_sum[0, 0] / n_elems)) + (loss_sum[0, 0] / n_elems)
    z_q = zp + (zq - zp)
    z_q = jnp.transpose(z_q, (0, 4, 1, 2, 3))
    return (z_q, loss, uniq[0, 0], idx)


# speedup vs baseline: 1.2348x; 1.0093x over previous
import functools

import jax
import jax.numpy as jnp
from jax import lax
from jax.experimental import pallas as pl
from jax.experimental.pallas import tpu as pltpu
from jax.experimental.pallas import tpu_sc as plsc

_N_E = 8192
_E_DIM = 256
_BETA = 0.25
_TB = 512
_NT = _N_E // _TB


def _dist_argmin_body(z_ref, w_ref, sz_ref, sw_ref,
                      idx_ref, loss_ref, loss_acc):
    i = pl.program_id(0)

    @pl.when(i == 0)
    def _init():
        loss_acc[0, 0] = jnp.float32(0.0)

    zb = z_ref[...]
    zb2 = zb + zb
    dot2 = lax.dot_general(zb2, w_ref[...], (((1,), (0,)), ((), ())),
                           preferred_element_type=jnp.float32)
    d = (sz_ref[...] + sw_ref[...]) - dot2
    rmin = jnp.min(d, axis=1, keepdims=True)
    ci = lax.broadcasted_iota(jnp.int32, (_TB, _N_E), 1)
    ridx = jnp.min(jnp.where(d == rmin, ci, jnp.int32(_N_E)),
                   axis=1, keepdims=True)

    idx_ref[...] = ridx
    loss_acc[0, 0] += jnp.sum(rmin)

    @pl.when(i == _NT - 1)
    def _fini():
        loss_ref[...] = jnp.full((1, 1), loss_acc[0, 0], jnp.float32)


def _dist_argmin(z_flat, wt, sz, sw):
    return pl.pallas_call(
        _dist_argmin_body,
        grid=(_NT,),
        in_specs=[
            pl.BlockSpec((_TB, _E_DIM), lambda i: (i, 0)),
            pl.BlockSpec((_E_DIM, _N_E), lambda i: (0, 0)),
            pl.BlockSpec((_TB, 1), lambda i: (i, 0)),
            pl.BlockSpec((1, _N_E), lambda i: (0, 0)),
        ],
        out_specs=[
            pl.BlockSpec((_TB, 1), lambda i: (i, 0)),
            pl.BlockSpec((1, 1), lambda i: (0, 0)),
        ],
        out_shape=[
            jax.ShapeDtypeStruct((_N_E, 1), jnp.int32),
            jax.ShapeDtypeStruct((1, 1), jnp.float32),
        ],
        scratch_shapes=[
            pltpu.SMEM((1, 1), jnp.float32),
        ],
    )(z_flat, wt, sz, sw)


_SC_INFO = plsc.get_sparse_core_info()
_NC = _SC_INFO.num_cores
_NS = _SC_INFO.num_subcores
_NW = _NC * _NS
_BPW = _N_E // _NW
_CH = 128
_PC = _N_E // _NS
_L = 16


def _sc_gather_body(table_hbm, idx_hbm, out_hbm, pres_hbm,
                    idx_a, idx_b, rows_v, idx_w, ones_v, zeros_v,
                    shared_pres, sem, sem2):
    cid = lax.axis_index("c")
    sid = lax.axis_index("s")
    wid = sid * _NC + cid
    base = wid * _BPW
    pltpu.sync_copy(idx_hbm.at[pl.ds(base, _CH)], idx_a)
    pltpu.sync_copy(idx_hbm.at[pl.ds(base + _CH, _CH)], idx_b)
    cp1 = pltpu.async_copy(table_hbm.at[idx_a], rows_v.at[pl.ds(0, _CH)], sem)
    cp2 = pltpu.async_copy(table_hbm.at[idx_b], rows_v.at[pl.ds(_CH, _CH)], sem)

    @pl.when(cid == 0)
    def _prep():
        for k in range(_PC // _L):
            zeros_v[pl.ds(k * _L, _L)] = jnp.zeros((_L,), jnp.int32)
        pltpu.sync_copy(zeros_v, shared_pres.at[pl.ds(sid * _PC, _PC)])
        for r in range(4):
            pltpu.sync_copy(idx_hbm.at[pl.ds(sid * _PC + r * _CH, _CH)],
                            idx_w.at[r])
        for k in range(_CH // _L):
            ones_v[pl.ds(k * _L, _L)] = jnp.ones((_L,), jnp.int32)

    plsc.subcore_barrier()

    @pl.when(cid == 0)
    def _scatter():
        cps = [pltpu.async_copy(ones_v, shared_pres.at[idx_w.at[r]], sem2,
                                add=True) for r in range(4)]
        for cp in cps:
            cp.wait()

    plsc.subcore_barrier()

    @pl.when(cid == 0)
    def _dump():
        pltpu.sync_copy(shared_pres.at[pl.ds(sid * _PC, _PC)],
                        pres_hbm.at[pl.ds(sid * _PC, _PC)])

    cp1.wait()
    cp2.wait()
    pltpu.sync_copy(rows_v, out_hbm.at[pl.ds(base, _BPW)])


_sc_gather = functools.partial(
    pl.kernel,
    mesh=plsc.VectorSubcoreMesh(core_axis_name="c", subcore_axis_name="s"),
    out_type=[
        jax.ShapeDtypeStruct((_N_E, _E_DIM), jnp.float32),
        jax.ShapeDtypeStruct((_N_E,), jnp.int32),
    ],
    scratch_types=[
        pltpu.VMEM((_CH,), jnp.int32),
        pltpu.VMEM((_CH,), jnp.int32),
        pltpu.VMEM((_BPW, _E_DIM), jnp.float32),
        pltpu.VMEM((4, _CH), jnp.int32),
        pltpu.VMEM((_CH,), jnp.int32),
        pltpu.VMEM((_PC,), jnp.int32),
        pltpu.VMEM_SHARED((_N_E,), jnp.int32),
        pltpu.SemaphoreType.DMA,
        pltpu.SemaphoreType.DMA,
    ],
)(_sc_gather_body)


def _uniq_body(pres_ref, uniq_ref):
    uniq_ref[...] = jnp.sum(jnp.minimum(pres_ref[...], 1), keepdims=True)


def _uniq_count(pres2d):
    return pl.pallas_call(
        _uniq_body,
        out_shape=jax.ShapeDtypeStruct((1, 1), jnp.int32),
    )(pres2d)


def kernel(z, W):
    zp = jnp.transpose(z, (0, 2, 3, 4, 1))
    z_flat = zp.reshape(-1, _E_DIM)
    sz = jnp.sum(z_flat ** 2, axis=1, keepdims=True)
    sw = jnp.sum(W ** 2, axis=1)
    idx2, loss_sum = _dist_argmin(z_flat, W.T, sz, sw.reshape(1, _N_E))
    idx = idx2.reshape(_N_E)

    zq_rows, pres = _sc_gather(W, idx)
    zq = zq_rows.reshape(zp.shape)
    uniq = _uniq_count(pres.reshape(1, _N_E))

    n_elems = z_flat.shape[0] * _E_DIM
    loss = (_BETA * (loss_sum[0, 0] / n_elems)) + (loss_sum[0, 0] / n_elems)
    z_q = zp + (zq - zp)
    z_q = jnp.transpose(z_q, (0, 4, 1, 2, 3))
    return (z_q, loss, uniq[0, 0], idx)
